# baseline (speedup 1.0000x reference)
import functools

import jax
import jax.numpy as jnp
from jax import lax
from jax.experimental import pallas as pl
from jax.experimental.pallas import tpu as pltpu
from jax.experimental.pallas import tpu_sc as plsc

_N, _C, _D, _GH, _GW = 4, 12, 8, 16, 16
_H = 512
_W = 512

_NUM_CORES = 2
_NUM_SUBCORES = 16
_NW = _NUM_CORES * _NUM_SUBCORES
_ROWS = _N * _H
_ROWS_PER_TEC = _ROWS // _NW
_SR = 4
_N_STAGES = _ROWS_PER_TEC // _SR
_ZS = _GW * _C + 1
_TBL = _D * _ZS
_RT = 2 * _TBL


def _tent_matrix(n_out, n_in):
    coord = jnp.arange(n_out, dtype=jnp.float32) / (n_out - 1) * 2.0 - 1.0
    i = jnp.clip((coord + 1.0) * 0.5 * (n_in - 1), 0.0, float(n_in - 1))
    k = jnp.arange(n_in, dtype=jnp.float32)
    return jnp.maximum(0.0, 1.0 - jnp.abs(i[:, None] - k[None, :]))


def _x_tables():
    coord = jnp.arange(_W, dtype=jnp.float32) / (_W - 1) * 2.0 - 1.0
    ix = jnp.clip((coord + 1.0) * 0.5 * (_GW - 1), 0.0, float(_GW - 1))
    ix0 = jnp.floor(ix)
    ix1 = jnp.minimum(ix0 + 1.0, float(_GW - 1))
    wx = ix - ix0
    par = (jnp.arange(_W) % 2) * _TBL
    b0 = (ix0 * _C).astype(jnp.int32) + par.astype(jnp.int32)
    dx = ((ix1 - ix0) * _C).astype(jnp.int32)
    return wx, b0, dx


def _rowtab_kernel(by_ref, grid_ref, out_ref):
    tmp = jnp.dot(by_ref[...], grid_ref[0],
                  preferred_element_type=jnp.float32)
    zw = _GW * _C

    def bits(x):
        b = jax.lax.bitcast_convert_type(x.astype(jnp.bfloat16), jnp.uint16)
        return b.astype(jnp.uint32)

    for z in range(_D):
        a = tmp[:, z * zw:(z + 1) * zw]
        zn = min(z + 1, _D - 1)
        b = tmp[:, zn * zw:(zn + 1) * zw]
        word = (bits(a) | (bits(b) << 16)).astype(jnp.int32)
        out_ref[0, :, z * _ZS:z * _ZS + zw] = word
        out_ref[0, :, _TBL + z * _ZS:_TBL + z * _ZS + zw] = word


def _make_row_tables(grid):
    gr = grid.transpose(0, 3, 2, 4, 1).reshape(_N, _GH, _D * _GW * _C)
    by = _tent_matrix(_H, _GH)
    r = pl.pallas_call(
        _rowtab_kernel,
        grid=(_N,),
        in_specs=[
            pl.BlockSpec((_H, _GH), lambda n: (0, 0)),
            pl.BlockSpec((1, _GH, _D * _GW * _C), lambda n: (n, 0, 0)),
        ],
        out_specs=pl.BlockSpec((1, _H, _RT), lambda n: (n, 0, 0)),
        out_shape=jax.ShapeDtypeStruct((_N, _H, _RT), jnp.int32),
    )(by, gr)
    return r.reshape(_ROWS * _RT)


def _sc_body(r_hbm, gm_hbm, wx_hbm, b0_hbm, dx_hbm, out_hbm,
             wx_v, b0_v, dx_v, r_v0, g_v0, o_v0, r_v1, g_v1, o_v1,
             in_s0, in_s1, out_s0, out_s1):
    wid = lax.axis_index("s") * _NUM_CORES + lax.axis_index("c")
    row_base = wid * _ROWS_PER_TEC

    pltpu.sync_copy(wx_hbm, wx_v)
    pltpu.sync_copy(b0_hbm, b0_v)
    pltpu.sync_copy(dx_hbm, dx_v)

    def start_in(s, r_v, g_v, sem):
        r0 = row_base + s * _SR
        pltpu.make_async_copy(
            r_hbm.at[pl.ds(r0 * _RT, _SR * _RT)], r_v, sem).start()
        pltpu.make_async_copy(
            gm_hbm.at[pl.ds(r0 * _W, _SR * _W)], g_v, sem).start()

    def wait_in(r_v, g_v, sem):
        pltpu.make_async_copy(
            r_hbm.at[pl.ds(0, _SR * _RT)], r_v, sem).wait()
        pltpu.make_async_copy(
            gm_hbm.at[pl.ds(0, _SR * _W)], g_v, sem).wait()

    def start_out(s, o_v, sem):
        r0 = row_base + s * _SR
        n = r0 // _H
        h = r0 - n * _H
        for c in range(_C):
            pltpu.make_async_copy(
                o_v.at[c], out_hbm.at[n, c, pl.ds(h, _SR)], sem).start()

    def wait_out(o_v, sem):
        for c in range(_C):
            pltpu.make_async_copy(
                o_v.at[c], out_hbm.at[0, c, pl.ds(0, _SR)], sem).wait()

    def compute(g_v, r_v, o_v):
        def group(j, t):
            off = t * 16
            joff = j * _RT
            gmv = g_v[pl.ds(j * _W + off, 16)]
            wxv = wx_v[pl.ds(off, 16)]
            b0s = b0_v[pl.ds(off, 16)]
            dxs = dx_v[pl.ds(off, 16)]
            z = gmv * 2.0 - 1.0
            iz = jnp.clip((z + 1.0) * 0.5 * (_D - 1), 0.0, float(_D - 1))
            iz0 = iz.astype(jnp.int32)
            wz = iz - iz0.astype(jnp.float32)
            b00 = iz0 * _ZS + b0s + joff
            b01 = b00 + dxs
            w00 = (1.0 - wz) * (1.0 - wxv)
            w01 = (1.0 - wz) * wxv
            w10 = wz * (1.0 - wxv)
            w11 = wz * wxv
            wp0 = plsc.pack(w00, w10, format=plsc.PackFormat.INTERLEAVED)
            wp1 = plsc.pack(w01, w11, format=plsc.PackFormat.INTERLEAVED)
            for c in range(_C):
                u0 = plsc.load_gather(r_v, [b00 + c])
                u1 = plsc.load_gather(r_v, [b01 + c])
                v0 = plsc.bitcast(u0, jnp.bfloat16)
                v1 = plsc.bitcast(u1, jnp.bfloat16)
                s = v0 * wp0 + v1 * wp1
                lo, hi = plsc.unpack(s, format=plsc.PackFormat.INTERLEAVED)
                o_v[c, j, pl.ds(off, 16)] = lo + hi

        for j in range(_SR):
            @plsc.parallel_loop(0, _W // 16, 1, unroll=2)
            def _(t, j=j):
                group(j, t)

    start_in(0, r_v0, g_v0, in_s0)
    start_in(1, r_v1, g_v1, in_s1)

    def super_stage(i, carry):
        s0 = 2 * i

        def half(s, r_v, g_v, o_v, in_sem, out_sem):
            with jax.named_scope("wait_prev_out"):
                @pl.when(s >= 2)
                def _():
                    wait_out(o_v, out_sem)
            wait_in(r_v, g_v, in_sem)
            compute(g_v, r_v, o_v)
            start_out(s, o_v, out_sem)

            @pl.when(s + 2 < _N_STAGES)
            def _():
                start_in(s + 2, r_v, g_v, in_sem)

        half(s0, r_v0, g_v0, o_v0, in_s0, out_s0)
        half(s0 + 1, r_v1, g_v1, o_v1, in_s1, out_s1)
        return carry

    lax.fori_loop(0, _N_STAGES // 2, super_stage, 0)
    wait_out(o_v0, out_s0)
    wait_out(o_v1, out_s1)


def _sc_slice(r_flat, gm_flat, wx, b0, dx):
    mesh = plsc.VectorSubcoreMesh(core_axis_name="c", subcore_axis_name="s",
                                  num_cores=_NUM_CORES,
                                  num_subcores=_NUM_SUBCORES)
    return pl.kernel(
        _sc_body,
        out_type=jax.ShapeDtypeStruct((_N, _C, _H, _W), jnp.float32),
        mesh=mesh,
        compiler_params=pltpu.CompilerParams(needs_layout_passes=False),
        scratch_types=[
            pltpu.VMEM((_W,), jnp.float32),
            pltpu.VMEM((_W,), jnp.int32),
            pltpu.VMEM((_W,), jnp.int32),
            pltpu.VMEM((_SR * _RT,), jnp.int32),
            pltpu.VMEM((_SR * _W,), jnp.float32),
            pltpu.VMEM((_C, _SR, _W), jnp.float32),
            pltpu.VMEM((_SR * _RT,), jnp.int32),
            pltpu.VMEM((_SR * _W,), jnp.float32),
            pltpu.VMEM((_C, _SR, _W), jnp.float32),
            pltpu.SemaphoreType.DMA,
            pltpu.SemaphoreType.DMA,
            pltpu.SemaphoreType.DMA,
            pltpu.SemaphoreType.DMA,
        ],
    )(r_flat, gm_flat, wx, b0, dx)


@jax.jit
def kernel(grid, guidemap):
    r_flat = _make_row_tables(grid)
    gm_flat = guidemap.reshape(_ROWS * _W)
    wx, b0, dx = _x_tables()
    return _sc_slice(r_flat, gm_flat, wx, b0, dx)


# speedup vs baseline: 1.0954x; 1.0954x over previous
import functools

import jax
import jax.numpy as jnp
from jax import lax
from jax.experimental import pallas as pl
from jax.experimental.pallas import tpu as pltpu
from jax.experimental.pallas import tpu_sc as plsc

_N, _C, _D, _GH, _GW = 4, 12, 8, 16, 16
_H = 512
_W = 512

_NUM_CORES = 2
_NUM_SUBCORES = 16
_NW = _NUM_CORES * _NUM_SUBCORES
_ROWS = _N * _H
_ROWS_PER_TEC = _ROWS // _NW
_SR = 4
_N_STAGES = _ROWS_PER_TEC // _SR
_ZS = _GW * _C + 1
_TBL = _D * _ZS


def _tent_matrix(n_out, n_in):
    coord = jnp.arange(n_out, dtype=jnp.float32) / (n_out - 1) * 2.0 - 1.0
    i = jnp.clip((coord + 1.0) * 0.5 * (n_in - 1), 0.0, float(n_in - 1))
    k = jnp.arange(n_in, dtype=jnp.float32)
    return jnp.maximum(0.0, 1.0 - jnp.abs(i[:, None] - k[None, :]))


def _x_tables():
    coord = jnp.arange(_W, dtype=jnp.float32) / (_W - 1) * 2.0 - 1.0
    ix = jnp.clip((coord + 1.0) * 0.5 * (_GW - 1), 0.0, float(_GW - 1))
    ix0 = jnp.floor(ix)
    ix1 = jnp.minimum(ix0 + 1.0, float(_GW - 1))
    wx = ix - ix0
    b0 = (ix0 * _C).astype(jnp.int32)
    dx = ((ix1 - ix0) * _C).astype(jnp.int32)
    return wx, b0, dx


def _rowtab_kernel(by_ref, grid_ref, out_ref):
    tmp = jnp.dot(by_ref[...], grid_ref[0],
                  preferred_element_type=jnp.float32)
    zw = _GW * _C

    def bits(x):
        b = jax.lax.bitcast_convert_type(x.astype(jnp.bfloat16), jnp.uint16)
        return b.astype(jnp.uint32)

    for z in range(_D):
        a = tmp[:, z * zw:(z + 1) * zw]
        zn = min(z + 1, _D - 1)
        b = tmp[:, zn * zw:(zn + 1) * zw]
        word = (bits(a) | (bits(b) << 16)).astype(jnp.int32)
        out_ref[0, :, z * _ZS:z * _ZS + zw] = word


def _make_row_tables(grid):
    gr = grid.transpose(0, 3, 2, 4, 1).reshape(_N, _GH, _D * _GW * _C)
    by = _tent_matrix(_H, _GH)
    r = pl.pallas_call(
        _rowtab_kernel,
        grid=(_N,),
        in_specs=[
            pl.BlockSpec((_H, _GH), lambda n: (0, 0)),
            pl.BlockSpec((1, _GH, _D * _GW * _C), lambda n: (n, 0, 0)),
        ],
        out_specs=pl.BlockSpec((1, _H, _TBL), lambda n: (n, 0, 0)),
        out_shape=jax.ShapeDtypeStruct((_N, _H, _TBL), jnp.int32),
    )(by, gr)
    return r.reshape(_ROWS * _TBL)


def _sc_body(r_hbm, gm_hbm, wx_hbm, b0_hbm, dx_hbm, out_hbm,
             wx_v, b0_v, dx_v, r_v0, g_v0, o_v0, r_v1, g_v1, o_v1,
             in_s0, in_s1, out_s0, out_s1):
    wid = lax.axis_index("s") * _NUM_CORES + lax.axis_index("c")
    row_base = wid * _ROWS_PER_TEC

    pltpu.sync_copy(wx_hbm, wx_v)
    pltpu.sync_copy(b0_hbm, b0_v)
    pltpu.sync_copy(dx_hbm, dx_v)

    def start_in(s, r_v, g_v, sem):
        r0 = row_base + s * _SR
        pltpu.make_async_copy(
            r_hbm.at[pl.ds(r0 * _TBL, _SR * _TBL)], r_v, sem).start()
        pltpu.make_async_copy(
            gm_hbm.at[pl.ds(r0 * _W, _SR * _W)], g_v, sem).start()

    def wait_in(r_v, g_v, sem):
        pltpu.make_async_copy(
            r_hbm.at[pl.ds(0, _SR * _TBL)], r_v, sem).wait()
        pltpu.make_async_copy(
            gm_hbm.at[pl.ds(0, _SR * _W)], g_v, sem).wait()

    def start_out(s, o_v, sem):
        r0 = row_base + s * _SR
        n = r0 // _H
        h = r0 - n * _H
        for c in range(_C):
            pltpu.make_async_copy(
                o_v.at[c], out_hbm.at[n, c, pl.ds(h, _SR)], sem).start()

    def wait_out(o_v, sem):
        for c in range(_C):
            pltpu.make_async_copy(
                o_v.at[c], out_hbm.at[0, c, pl.ds(0, _SR)], sem).wait()

    def compute(g_v, r_v, o_v):
        def group(j, t):
            off = t * 16
            joff = j * _TBL
            gmv = g_v[pl.ds(j * _W + off, 16)]
            wxv = wx_v[pl.ds(off, 16)]
            b0s = b0_v[pl.ds(off, 16)]
            dxs = dx_v[pl.ds(off, 16)]
            z = gmv * 2.0 - 1.0
            iz = jnp.clip((z + 1.0) * 0.5 * (_D - 1), 0.0, float(_D - 1))
            iz0 = iz.astype(jnp.int32)
            wz = iz - iz0.astype(jnp.float32)
            b00 = iz0 * _ZS + b0s + joff
            b01 = b00 + dxs
            w00 = (1.0 - wz) * (1.0 - wxv)
            w01 = (1.0 - wz) * wxv
            w10 = wz * (1.0 - wxv)
            w11 = wz * wxv
            wp0 = plsc.pack(w00, w10, format=plsc.PackFormat.INTERLEAVED)
            wp1 = plsc.pack(w01, w11, format=plsc.PackFormat.INTERLEAVED)
            for c in range(_C):
                u0 = plsc.load_gather(r_v, [b00 + c])
                u1 = plsc.load_gather(r_v, [b01 + c])
                v0 = plsc.bitcast(u0, jnp.bfloat16)
                v1 = plsc.bitcast(u1, jnp.bfloat16)
                s = v0 * wp0 + v1 * wp1
                lo, hi = plsc.unpack(s, format=plsc.PackFormat.INTERLEAVED)
                o_v[c, j, pl.ds(off, 16)] = lo + hi

        for j in range(_SR):
            @plsc.parallel_loop(0, _W // 16, 1, unroll=2)
            def _(t, j=j):
                group(j, t)

    start_in(0, r_v0, g_v0, in_s0)
    start_in(1, r_v1, g_v1, in_s1)

    def super_stage(i, carry):
        s0 = 2 * i

        def half(s, r_v, g_v, o_v, in_sem, out_sem):
            with jax.named_scope("wait_prev_out"):
                @pl.when(s >= 2)
                def _():
                    wait_out(o_v, out_sem)
            wait_in(r_v, g_v, in_sem)
            compute(g_v, r_v, o_v)
            start_out(s, o_v, out_sem)

            @pl.when(s + 2 < _N_STAGES)
            def _():
                start_in(s + 2, r_v, g_v, in_sem)

        half(s0, r_v0, g_v0, o_v0, in_s0, out_s0)
        half(s0 + 1, r_v1, g_v1, o_v1, in_s1, out_s1)
        return carry

    lax.fori_loop(0, _N_STAGES // 2, super_stage, 0)
    wait_out(o_v0, out_s0)
    wait_out(o_v1, out_s1)


def _sc_slice(r_flat, gm_flat, wx, b0, dx):
    mesh = plsc.VectorSubcoreMesh(core_axis_name="c", subcore_axis_name="s",
                                  num_cores=_NUM_CORES,
                                  num_subcores=_NUM_SUBCORES)
    return pl.kernel(
        _sc_body,
        out_type=jax.ShapeDtypeStruct((_N, _C, _H, _W), jnp.float32),
        mesh=mesh,
        compiler_params=pltpu.CompilerParams(needs_layout_passes=False),
        scratch_types=[
            pltpu.VMEM((_W,), jnp.float32),
            pltpu.VMEM((_W,), jnp.int32),
            pltpu.VMEM((_W,), jnp.int32),
            pltpu.VMEM((_SR * _TBL,), jnp.int32),
            pltpu.VMEM((_SR * _W,), jnp.float32),
            pltpu.VMEM((_C, _SR, _W), jnp.float32),
            pltpu.VMEM((_SR * _TBL,), jnp.int32),
            pltpu.VMEM((_SR * _W,), jnp.float32),
            pltpu.VMEM((_C, _SR, _W), jnp.float32),
            pltpu.SemaphoreType.DMA,
            pltpu.SemaphoreType.DMA,
            pltpu.SemaphoreType.DMA,
            pltpu.SemaphoreType.DMA,
        ],
    )(r_flat, gm_flat, wx, b0, dx)


@jax.jit
def kernel(grid, guidemap):
    r_flat = _make_row_tables(grid)
    gm_flat = guidemap.reshape(_ROWS * _W)
    wx, b0, dx = _x_tables()
    return _sc_slice(r_flat, gm_flat, wx, b0, dx)
